# baseline (speedup 1.0000x reference)
import functools

import jax
import jax.numpy as jnp
from jax import lax
from jax.experimental import pallas as pl
from jax.experimental.pallas import tpu as pltpu
from jax.experimental.pallas import tpu_sc as plsc

B, T, D = 4, 8192, 768
H = D // 4
DH = D // 2
MAX_OBS = 2000
GAP_ROWS = 128
N = B * T
N2 = 2 * N

_NEG = -(2 ** 30)


def _erf(z):
    a1, a2, a3, a4, a5 = (0.254829592, -0.284496736, 1.421413741,
                          -1.453152027, 1.061405429)
    p = 0.3275911
    s = jnp.sign(z)
    az = jnp.abs(z)
    t = 1.0 / (1.0 + p * az)
    poly = ((((a5 * t + a4) * t + a3) * t + a2) * t + a1) * t
    return s * (1.0 - poly * jnp.exp(-az * az))


def _index_kernel(mask_ref, w1_ref, b1_ref, w2_ref, b2_ref,
                  obs_ref, gap_ref, tbl_ref):
    valid = mask_ref[...]
    csum = valid
    s = 1
    while s < T:
        shifted = jnp.concatenate(
            [jnp.zeros((B, s), jnp.int32), csum[:, :T - s]], axis=1)
        csum = csum + shifted
        s *= 2
    obs_ref[...] = jnp.clip(csum - 1, 0, MAX_OBS - 1)

    pos = lax.broadcasted_iota(jnp.int32, (B, T), 1)
    lastv = jnp.where(valid > 0, pos, _NEG)
    s = 1
    while s < T:
        shifted = jnp.concatenate(
            [jnp.full((B, s), _NEG, jnp.int32), lastv[:, :T - s]], axis=1)
        lastv = jnp.maximum(lastv, shifted)
        s *= 2
    gap = jnp.where(lastv < 0, 0, jnp.minimum(pos - lastv, 100))
    gap_ref[...] = gap + MAX_OBS

    gmax = jnp.max(gap).astype(jnp.float32)
    k = lax.broadcasted_iota(jnp.int32, (GAP_ROWS, H), 0).astype(jnp.float32)
    g = k / (gmax + 1e-8)
    z = g * w1_ref[...] + b1_ref[...]
    h1 = 0.5 * z * (1.0 + _erf(z * 0.7071067811865476))
    tbl_ref[...] = (jnp.dot(h1, w2_ref[...], preferred_element_type=jnp.float32)
                    + b2_ref[...])


_index_call = pl.pallas_call(
    _index_kernel,
    out_shape=(
        jax.ShapeDtypeStruct((B, T), jnp.int32),
        jax.ShapeDtypeStruct((B, T), jnp.int32),
        jax.ShapeDtypeStruct((GAP_ROWS, DH), jnp.float32),
    ),
)

CHUNK = 128


@functools.lru_cache(maxsize=None)
def _make_gather_kernel():
    info = plsc.get_sparse_core_info()
    nc, ns = info.num_cores, info.num_subcores
    nw = nc * ns
    per_w = N2 // nw
    nchunk = per_w // CHUNK
    mesh = plsc.VectorSubcoreMesh(core_axis_name="c", subcore_axis_name="s")

    @functools.partial(
        pl.kernel,
        mesh=mesh,
        out_type=jax.ShapeDtypeStruct((N2, DH), jnp.float32),
        scratch_types=[
            pltpu.VMEM((CHUNK,), jnp.int32),
            pltpu.VMEM((CHUNK,), jnp.int32),
            pltpu.VMEM((CHUNK, DH), jnp.float32),
            pltpu.VMEM((CHUNK, DH), jnp.float32),
            pltpu.SemaphoreType.DMA,
            pltpu.SemaphoreType.DMA,
            pltpu.SemaphoreType.DMA,
            pltpu.SemaphoreType.DMA,
        ],
    )
    def _gather_kernel(tbl_hbm, idx_hbm, out_hbm,
                       idx0, idx1, rows0, rows1, g0, g1, s0, s1):
        wid = lax.axis_index("s") * nc + lax.axis_index("c")
        base = wid * per_w
        idxb = (idx0, idx1)
        rows = (rows0, rows1)
        gsem = (g0, g1)
        ssem = (s0, s1)

        def start_gather(i, b):
            off = base + i * CHUNK
            pltpu.sync_copy(idx_hbm.at[pl.ds(off, CHUNK)], idxb[b])
            return pltpu.async_copy(tbl_hbm.at[idxb[b]], rows[b], gsem[b])

        gh = [start_gather(0, 0), start_gather(1, 1)]
        sh = [None, None]
        for i in range(nchunk):
            b = i & 1
            gh[b].wait()
            off = base + i * CHUNK
            sh[b] = pltpu.async_copy(rows[b], out_hbm.at[pl.ds(off, CHUNK)],
                                     ssem[b])
            if i + 2 < nchunk:
                sh[b].wait()
                gh[b] = start_gather(i + 2, b)
        sh[0].wait()
        sh[1].wait()

    return _gather_kernel


def kernel(x, padding_mask, obs_table, W1, b1, W2, b2):
    mask_i32 = (~padding_mask).astype(jnp.int32)
    obs_idx, gap_idx, gap_tbl = _index_call(
        mask_i32, W1, b1.reshape(1, H), W2, b2.reshape(1, DH))
    idx = jnp.stack([obs_idx.reshape(-1), gap_idx.reshape(-1)],
                    axis=-1).reshape(-1)
    tbl = jnp.concatenate([obs_table, gap_tbl], axis=0)
    out = _make_gather_kernel()(tbl, idx)
    return out.reshape(B, T, D)


# speedup vs baseline: 1.0046x; 1.0046x over previous
import functools

import jax
import jax.numpy as jnp
from jax import lax
from jax.experimental import pallas as pl
from jax.experimental.pallas import tpu as pltpu
from jax.experimental.pallas import tpu_sc as plsc

B, T, D = 4, 8192, 768
H = D // 4
DH = D // 2
MAX_OBS = 2000
GAP_ROWS = 128
N = B * T
N2 = 2 * N

_NEG = -(2 ** 30)


def _erf(z):
    a1, a2, a3, a4, a5 = (0.254829592, -0.284496736, 1.421413741,
                          -1.453152027, 1.061405429)
    p = 0.3275911
    s = jnp.sign(z)
    az = jnp.abs(z)
    t = 1.0 / (1.0 + p * az)
    poly = ((((a5 * t + a4) * t + a3) * t + a2) * t + a1) * t
    return s * (1.0 - poly * jnp.exp(-az * az))


def _index_kernel(mask_ref, w1_ref, b1_ref, w2_ref, b2_ref,
                  obs_ref, gap_ref, tbl_ref):
    valid = mask_ref[...]
    csum = valid
    s = 1
    while s < T:
        shifted = jnp.concatenate(
            [jnp.zeros((B, s), jnp.int32), csum[:, :T - s]], axis=1)
        csum = csum + shifted
        s *= 2
    obs_ref[...] = jnp.clip(csum - 1, 0, MAX_OBS - 1)

    pos = lax.broadcasted_iota(jnp.int32, (B, T), 1)
    lastv = jnp.where(valid > 0, pos, _NEG)
    s = 1
    while s < T:
        shifted = jnp.concatenate(
            [jnp.full((B, s), _NEG, jnp.int32), lastv[:, :T - s]], axis=1)
        lastv = jnp.maximum(lastv, shifted)
        s *= 2
    gap = jnp.where(lastv < 0, 0, jnp.minimum(pos - lastv, 100))
    gap_ref[...] = gap + MAX_OBS

    gmax = jnp.max(gap).astype(jnp.float32)
    k = lax.broadcasted_iota(jnp.int32, (GAP_ROWS, H), 0).astype(jnp.float32)
    g = k / (gmax + 1e-8)
    z = g * w1_ref[...] + b1_ref[...]
    h1 = 0.5 * z * (1.0 + _erf(z * 0.7071067811865476))
    tbl_ref[...] = (jnp.dot(h1, w2_ref[...], preferred_element_type=jnp.float32)
                    + b2_ref[...])


_index_call = pl.pallas_call(
    _index_kernel,
    out_shape=(
        jax.ShapeDtypeStruct((B, T), jnp.int32),
        jax.ShapeDtypeStruct((B, T), jnp.int32),
        jax.ShapeDtypeStruct((GAP_ROWS, DH), jnp.float32),
    ),
)

CHUNK = 128


@functools.lru_cache(maxsize=None)
def _make_gather_kernel():
    info = plsc.get_sparse_core_info()
    nc, ns = info.num_cores, info.num_subcores
    nw = nc * ns
    per_w = N2 // nw
    nchunk = per_w // CHUNK
    mesh = plsc.VectorSubcoreMesh(core_axis_name="c", subcore_axis_name="s")

    @functools.partial(
        pl.kernel,
        mesh=mesh,
        out_type=jax.ShapeDtypeStruct((N2, DH), jnp.float32),
        scratch_types=[
            pltpu.VMEM((per_w,), jnp.int32),
            pltpu.VMEM((CHUNK, DH), jnp.float32),
            pltpu.VMEM((CHUNK, DH), jnp.float32),
            pltpu.SemaphoreType.DMA,
            pltpu.SemaphoreType.DMA,
            pltpu.SemaphoreType.DMA,
            pltpu.SemaphoreType.DMA,
        ],
    )
    def _gather_kernel(tbl_hbm, idx_hbm, out_hbm,
                       idx_all, rows0, rows1, g0, g1, s0, s1):
        wid = lax.axis_index("s") * nc + lax.axis_index("c")
        base = wid * per_w
        rows = (rows0, rows1)
        gsem = (g0, g1)
        ssem = (s0, s1)
        pltpu.sync_copy(idx_hbm.at[pl.ds(base, per_w)], idx_all)

        def start_gather(i, b):
            return pltpu.async_copy(
                tbl_hbm.at[idx_all.at[pl.ds(i * CHUNK, CHUNK)]],
                rows[b], gsem[b])

        gh = [start_gather(0, 0), start_gather(1, 1)]
        sh = [None, None]
        for i in range(nchunk):
            b = i & 1
            gh[b].wait()
            off = base + i * CHUNK
            sh[b] = pltpu.async_copy(rows[b], out_hbm.at[pl.ds(off, CHUNK)],
                                     ssem[b])
            if i + 2 < nchunk:
                sh[b].wait()
                gh[b] = start_gather(i + 2, b)
        sh[0].wait()
        sh[1].wait()

    return _gather_kernel


def kernel(x, padding_mask, obs_table, W1, b1, W2, b2):
    mask_i32 = (~padding_mask).astype(jnp.int32)
    obs_idx, gap_idx, gap_tbl = _index_call(
        mask_i32, W1, b1.reshape(1, H), W2, b2.reshape(1, DH))
    idx = jnp.stack([obs_idx.reshape(-1), gap_idx.reshape(-1)],
                    axis=-1).reshape(-1)
    tbl = jnp.concatenate([obs_table, gap_tbl], axis=0)
    out = _make_gather_kernel()(tbl, idx)
    return out.reshape(B, T, D)


# speedup vs baseline: 2.6403x; 2.6282x over previous
import functools

import jax
import jax.numpy as jnp
from jax import lax
from jax.experimental import pallas as pl
from jax.experimental.pallas import tpu as pltpu
from jax.experimental.pallas import tpu_sc as plsc

B, T, D = 4, 8192, 768
H = D // 4
DH = D // 2
NLANE = DH // 16
MAX_OBS = 2000
GAP_ROWS = 104
N = B * T
CH = 16
WIN = 24

_NEG = -(2 ** 30)


def _erf(z):
    a1, a2, a3, a4, a5 = (0.254829592, -0.284496736, 1.421413741,
                          -1.453152027, 1.061405429)
    p = 0.3275911
    s = jnp.sign(z)
    az = jnp.abs(z)
    t = 1.0 / (1.0 + p * az)
    poly = ((((a5 * t + a4) * t + a3) * t + a2) * t + a1) * t
    return s * (1.0 - poly * jnp.exp(-az * az))


def _index_kernel(mask_ref, w1_ref, b1_ref, w2_ref, b2_ref,
                  pack_ref, tbl_ref):
    valid = mask_ref[...]
    csum = valid
    s = 1
    while s < T:
        shifted = jnp.concatenate(
            [jnp.zeros((B, s), jnp.int32), csum[:, :T - s]], axis=1)
        csum = csum + shifted
        s *= 2
    obs = jnp.clip(csum - 1, 0, MAX_OBS - 1)

    pos = lax.broadcasted_iota(jnp.int32, (B, T), 1)
    lastv = jnp.where(valid > 0, pos, _NEG)
    s = 1
    while s < T:
        shifted = jnp.concatenate(
            [jnp.full((B, s), _NEG, jnp.int32), lastv[:, :T - s]], axis=1)
        lastv = jnp.maximum(lastv, shifted)
        s *= 2
    gap = jnp.where(lastv < 0, 0, jnp.minimum(pos - lastv, 100))

    pos_in = pos & (CH - 1)
    f = jnp.where(pos_in == 0, obs, -1)
    s = 1
    while s < CH:
        shifted = jnp.concatenate(
            [jnp.full((B, s), -1, jnp.int32), f[:, :T - s]], axis=1)
        f = jnp.maximum(f, jnp.where(pos_in >= s, shifted, -1))
        s *= 2
    lo8 = jnp.minimum(f & -8, MAX_OBS - WIN)
    oloc = obs - lo8
    pack_ref[...] = oloc | (gap << 6) | (lo8 << 16)

    gmax = jnp.max(gap).astype(jnp.float32)
    k = lax.broadcasted_iota(jnp.int32, (GAP_ROWS, H), 0).astype(jnp.float32)
    g = k / (gmax + 1e-8)
    z = g * w1_ref[...] + b1_ref[...]
    h1 = 0.5 * z * (1.0 + _erf(z * 0.7071067811865476))
    tbl_ref[...] = (jnp.dot(h1, w2_ref[...], preferred_element_type=jnp.float32)
                    + b2_ref[...])


_index_call = pl.pallas_call(
    _index_kernel,
    out_shape=(
        jax.ShapeDtypeStruct((B, T), jnp.int32),
        jax.ShapeDtypeStruct((GAP_ROWS, DH), jnp.float32),
    ),
)


@functools.lru_cache(maxsize=None)
def _make_expand_kernel():
    info = plsc.get_sparse_core_info()
    nc, ns = info.num_cores, info.num_subcores
    nw = nc * ns
    tok_w = N // nw
    nch = tok_w // CH
    mesh = plsc.VectorSubcoreMesh(core_axis_name="c", subcore_axis_name="s")

    @functools.partial(
        pl.kernel,
        mesh=mesh,
        out_type=jax.ShapeDtypeStruct((2 * N, DH), jnp.float32),
        scratch_types=[
            pltpu.VMEM((tok_w,), jnp.int32),
            pltpu.VMEM((GAP_ROWS, DH), jnp.float32),
            pltpu.VMEM((WIN, DH), jnp.float32),
            pltpu.VMEM((WIN, DH), jnp.float32),
            pltpu.VMEM((2 * CH, DH), jnp.float32),
            pltpu.VMEM((2 * CH, DH), jnp.float32),
            pltpu.SemaphoreType.DMA,
            pltpu.SemaphoreType.DMA,
            pltpu.SemaphoreType.DMA,
            pltpu.SemaphoreType.DMA,
        ],
    )
    def _expand_kernel(obs_hbm, gap_hbm, pidx_hbm, out_hbm,
                       pidx, gapt, win0, win1, ob0, ob1, w0, w1, o0, o1):
        wid = lax.axis_index("s") * nc + lax.axis_index("c")
        tbase = wid * tok_w
        obase = wid * (2 * tok_w)
        wins = (win0, win1)
        obuf = (ob0, ob1)
        wsem = (w0, w1)
        osem = (o0, o1)

        pltpu.sync_copy(pidx_hbm.at[pl.ds(tbase, tok_w)], pidx)
        pltpu.sync_copy(gap_hbm, gapt)

        def start_win(ci, b):
            vec = pidx[pl.ds(ci * CH, 16)]
            lo8 = pl.multiple_of(lax.shift_right_logical(vec[0], 16), 8)
            pltpu.async_copy(obs_hbm.at[pl.ds(lo8, WIN)], wins[b], wsem[b])

        def wait_win(b):
            pltpu.make_async_copy(obs_hbm.at[pl.ds(0, WIN)],
                                  wins[b], wsem[b]).wait()

        def wait_out(b):
            pltpu.make_async_copy(obuf[b], out_hbm.at[pl.ds(0, 2 * CH)],
                                  osem[b]).wait()

        start_win(0, 0)
        start_win(1, 1)

        def process(i, ci, b):
            wait_win(b)

            @pl.when(i > 0)
            def _():
                wait_out(b)

            c0 = pidx[pl.ds(ci * CH, 16)]
            for j in range(CH):
                v = c0[j]
                oloc = v & 63
                g = lax.shift_right_logical(v, 6) & 127
                for k in range(NLANE):
                    sl = pl.ds(k * 16, 16)
                    obuf[b][2 * j, sl] = wins[b][oloc, sl]
                    obuf[b][2 * j + 1, sl] = gapt[g, sl]
            dst = pl.multiple_of(obase + ci * 2 * CH, 8)
            pltpu.async_copy(obuf[b], out_hbm.at[pl.ds(dst, 2 * CH)], osem[b])

            @pl.when(ci + 2 < nch)
            def _():
                start_win(ci + 2, b)

        def body(i, _):
            process(i, 2 * i, 0)
            process(i, 2 * i + 1, 1)
            return 0

        lax.fori_loop(0, nch // 2, body, 0)
        wait_out(0)
        wait_out(1)

    return _expand_kernel


def kernel(x, padding_mask, obs_table, W1, b1, W2, b2):
    mask_i32 = (~padding_mask).astype(jnp.int32)
    pack, gap_tbl = _index_call(
        mask_i32, W1, b1.reshape(1, H), W2, b2.reshape(1, DH))
    out = _make_expand_kernel()(obs_table, gap_tbl, pack.reshape(-1))
    return out.reshape(B, T, D)


# speedup vs baseline: 3.3964x; 1.2864x over previous
import functools

import jax
import jax.numpy as jnp
from jax import lax
from jax.experimental import pallas as pl
from jax.experimental.pallas import tpu as pltpu
from jax.experimental.pallas import tpu_sc as plsc

B, T, D = 4, 8192, 768
H = D // 4
DH = D // 2
NLANE = DH // 16
MAX_OBS = 2000
GAP_ROWS = 104
N = B * T
CH = 32
WIN = 40

_NEG = -(2 ** 30)


def _erf(z):
    a1, a2, a3, a4, a5 = (0.254829592, -0.284496736, 1.421413741,
                          -1.453152027, 1.061405429)
    p = 0.3275911
    s = jnp.sign(z)
    az = jnp.abs(z)
    t = 1.0 / (1.0 + p * az)
    poly = ((((a5 * t + a4) * t + a3) * t + a2) * t + a1) * t
    return s * (1.0 - poly * jnp.exp(-az * az))


def _index_kernel(mask_ref, w1_ref, b1_ref, w2_ref, b2_ref,
                  pack_ref, tbl_ref):
    valid = mask_ref[...]
    csum = valid
    s = 1
    while s < T:
        shifted = jnp.concatenate(
            [jnp.zeros((B, s), jnp.int32), csum[:, :T - s]], axis=1)
        csum = csum + shifted
        s *= 2
    obs = jnp.clip(csum - 1, 0, MAX_OBS - 1)

    pos = lax.broadcasted_iota(jnp.int32, (B, T), 1)
    lastv = jnp.where(valid > 0, pos, _NEG)
    s = 1
    while s < T:
        shifted = jnp.concatenate(
            [jnp.full((B, s), _NEG, jnp.int32), lastv[:, :T - s]], axis=1)
        lastv = jnp.maximum(lastv, shifted)
        s *= 2
    gap = jnp.where(lastv < 0, 0, jnp.minimum(pos - lastv, 100))

    pos_in = pos & (CH - 1)
    f = jnp.where(pos_in == 0, obs, -1)
    s = 1
    while s < CH:
        shifted = jnp.concatenate(
            [jnp.full((B, s), -1, jnp.int32), f[:, :T - s]], axis=1)
        f = jnp.maximum(f, jnp.where(pos_in >= s, shifted, -1))
        s *= 2
    lo8 = jnp.minimum(f & -8, MAX_OBS - WIN)
    oloc = obs - lo8
    pack_ref[...] = oloc | (gap << 6) | (lo8 << 16)

    gmax = jnp.max(gap).astype(jnp.float32)
    k = lax.broadcasted_iota(jnp.int32, (GAP_ROWS, H), 0).astype(jnp.float32)
    g = k / (gmax + 1e-8)
    z = g * w1_ref[...] + b1_ref[...]
    h1 = 0.5 * z * (1.0 + _erf(z * 0.7071067811865476))
    tbl_ref[...] = (jnp.dot(h1, w2_ref[...], preferred_element_type=jnp.float32)
                    + b2_ref[...])


_index_call = pl.pallas_call(
    _index_kernel,
    out_shape=(
        jax.ShapeDtypeStruct((B, T), jnp.int32),
        jax.ShapeDtypeStruct((GAP_ROWS, DH), jnp.float32),
    ),
)


@functools.lru_cache(maxsize=None)
def _make_expand_kernel():
    info = plsc.get_sparse_core_info()
    nc, ns = info.num_cores, info.num_subcores
    nw = nc * ns
    tok_w = N // nw
    nch = tok_w // CH
    mesh = plsc.VectorSubcoreMesh(core_axis_name="c", subcore_axis_name="s")

    wpb = T // tok_w

    @functools.partial(
        pl.kernel,
        mesh=mesh,
        out_type=jax.ShapeDtypeStruct((B, T, D), jnp.float32),
        scratch_types=[
            pltpu.VMEM((tok_w,), jnp.int32),
            pltpu.VMEM((GAP_ROWS, DH), jnp.float32),
            pltpu.VMEM((WIN, DH), jnp.float32),
            pltpu.VMEM((WIN, DH), jnp.float32),
            pltpu.VMEM((CH, D), jnp.float32),
            pltpu.VMEM((CH, D), jnp.float32),
            pltpu.SemaphoreType.DMA,
            pltpu.SemaphoreType.DMA,
            pltpu.SemaphoreType.DMA,
            pltpu.SemaphoreType.DMA,
        ],
    )
    def _expand_kernel(obs_hbm, gap_hbm, pidx_hbm, out_hbm,
                       pidx, gapt, win0, win1, ob0, ob1, w0, w1, o0, o1):
        wid = lax.axis_index("s") * nc + lax.axis_index("c")
        tbase = wid * tok_w
        bi = wid // wpb
        t0 = (wid % wpb) * tok_w
        wins = (win0, win1)
        obuf = (ob0, ob1)
        wsem = (w0, w1)
        osem = (o0, o1)

        pltpu.sync_copy(pidx_hbm.at[pl.ds(tbase, tok_w)], pidx)
        pltpu.sync_copy(gap_hbm, gapt)

        def start_win(ci, b):
            vec = pidx[pl.ds(ci * CH, 16)]
            lo8 = pl.multiple_of(lax.shift_right_logical(vec[0], 16), 8)
            pltpu.async_copy(obs_hbm.at[pl.ds(lo8, WIN)], wins[b], wsem[b])

        def wait_win(b):
            pltpu.make_async_copy(obs_hbm.at[pl.ds(0, WIN)],
                                  wins[b], wsem[b]).wait()

        def wait_out(b):
            pltpu.make_async_copy(obuf[b], out_hbm.at[0, pl.ds(0, CH)],
                                  osem[b]).wait()

        start_win(0, 0)
        start_win(1, 1)

        def process(i, ci, b):
            wait_win(b)

            @pl.when(i > 0)
            def _():
                wait_out(b)

            c0 = pidx[pl.ds(ci * CH, 16)]
            c1 = pidx[pl.ds(ci * CH + 16, 16)]
            for j in range(CH):
                v = c0[j] if j < 16 else c1[j - 16]
                oloc = v & 63
                g = lax.shift_right_logical(v, 6) & 127
                for k in range(NLANE):
                    obuf[b][j, pl.ds(k * 16, 16)] = wins[b][oloc, pl.ds(k * 16, 16)]
                    obuf[b][j, pl.ds(DH + k * 16, 16)] = gapt[g, pl.ds(k * 16, 16)]
            dst = pl.multiple_of(t0 + ci * CH, 8)
            pltpu.async_copy(obuf[b], out_hbm.at[bi, pl.ds(dst, CH)], osem[b])

            @pl.when(ci + 2 < nch)
            def _():
                start_win(ci + 2, b)

        def body(i, _):
            process(i, 2 * i, 0)
            process(i, 2 * i + 1, 1)
            return 0

        lax.fori_loop(0, nch // 2, body, 0)
        wait_out(0)
        wait_out(1)

    return _expand_kernel


def kernel(x, padding_mask, obs_table, W1, b1, W2, b2):
    mask_i32 = (~padding_mask).astype(jnp.int32)
    pack, gap_tbl = _index_call(
        mask_i32, W1, b1.reshape(1, H), W2, b2.reshape(1, DH))
    return _make_expand_kernel()(obs_table, gap_tbl, pack.reshape(-1))


# speedup vs baseline: 4.3020x; 1.2666x over previous
import functools

import jax
import jax.numpy as jnp
from jax import lax
from jax.experimental import pallas as pl
from jax.experimental.pallas import tpu as pltpu
from jax.experimental.pallas import tpu_sc as plsc

B, T, D = 4, 8192, 768
H = D // 4
DH = D // 2
NLANE = DH // 16
MAX_OBS = 2000
GAP_ROWS = 104
N = B * T
CH = 32
WIN = 40

_NEG = -(2 ** 30)


def _erf(z):
    a1, a2, a3, a4, a5 = (0.254829592, -0.284496736, 1.421413741,
                          -1.453152027, 1.061405429)
    p = 0.3275911
    s = jnp.sign(z)
    az = jnp.abs(z)
    t = 1.0 / (1.0 + p * az)
    poly = ((((a5 * t + a4) * t + a3) * t + a2) * t + a1) * t
    return s * (1.0 - poly * jnp.exp(-az * az))


def _index_kernel(mask_ref, w1_ref, b1_ref, w2_ref, b2_ref,
                  pack_ref, tbl_ref):
    valid = mask_ref[...]
    csum = valid
    s = 1
    while s < T:
        shifted = jnp.concatenate(
            [jnp.zeros((B, s), jnp.int32), csum[:, :T - s]], axis=1)
        csum = csum + shifted
        s *= 2
    obs = jnp.clip(csum - 1, 0, MAX_OBS - 1)

    pos = lax.broadcasted_iota(jnp.int32, (B, T), 1)
    lastv = jnp.where(valid > 0, pos, _NEG)
    s = 1
    while s < T:
        shifted = jnp.concatenate(
            [jnp.full((B, s), _NEG, jnp.int32), lastv[:, :T - s]], axis=1)
        lastv = jnp.maximum(lastv, shifted)
        s *= 2
    gap = jnp.where(lastv < 0, 0, jnp.minimum(pos - lastv, 100))

    pos_in = pos & (CH - 1)
    f = jnp.where(pos_in == 0, obs, -1)
    s = 1
    while s < CH:
        shifted = jnp.concatenate(
            [jnp.full((B, s), -1, jnp.int32), f[:, :T - s]], axis=1)
        f = jnp.maximum(f, jnp.where(pos_in >= s, shifted, -1))
        s *= 2
    lo8 = jnp.minimum(f & -8, MAX_OBS - WIN)
    oloc = obs - lo8
    pack_ref[...] = oloc | (gap << 6) | (lo8 << 16)

    gmax = jnp.max(gap).astype(jnp.float32)
    k = lax.broadcasted_iota(jnp.int32, (GAP_ROWS, H), 0).astype(jnp.float32)
    g = k / (gmax + 1e-8)
    z = g * w1_ref[...] + b1_ref[...]
    h1 = 0.5 * z * (1.0 + _erf(z * 0.7071067811865476))
    tbl_ref[...] = (jnp.dot(h1, w2_ref[...], preferred_element_type=jnp.float32)
                    + b2_ref[...])


_index_call = pl.pallas_call(
    _index_kernel,
    out_shape=(
        jax.ShapeDtypeStruct((B, T), jnp.int32),
        jax.ShapeDtypeStruct((GAP_ROWS, DH), jnp.float32),
    ),
)


@functools.lru_cache(maxsize=None)
def _make_expand_kernel():
    info = plsc.get_sparse_core_info()
    nc, ns = info.num_cores, info.num_subcores
    nw = nc * ns
    tok_w = N // nw
    nch = tok_w // CH
    mesh = plsc.VectorSubcoreMesh(core_axis_name="c", subcore_axis_name="s")

    wpb = T // tok_w

    @functools.partial(
        pl.kernel,
        mesh=mesh,
        out_type=jax.ShapeDtypeStruct((B, T, D), jnp.float32),
        scratch_types=[
            pltpu.VMEM((tok_w,), jnp.int32),
            pltpu.VMEM((GAP_ROWS, DH), jnp.float32),
            pltpu.VMEM((WIN, DH), jnp.float32),
            pltpu.VMEM((WIN, DH), jnp.float32),
            pltpu.VMEM((CH, D), jnp.float32),
            pltpu.VMEM((CH, D), jnp.float32),
            pltpu.SemaphoreType.DMA,
            pltpu.SemaphoreType.DMA,
            pltpu.SemaphoreType.DMA,
            pltpu.SemaphoreType.DMA,
        ],
    )
    def _expand_kernel(obs_hbm, gap_hbm, pidx_hbm, out_hbm,
                       pidx, gapt, win0, win1, ob0, ob1, w0, w1, o0, o1):
        wid = lax.axis_index("s") * nc + lax.axis_index("c")
        tbase = wid * tok_w
        bi = wid // wpb
        t0 = (wid % wpb) * tok_w
        wins = (win0, win1)
        obuf = (ob0, ob1)
        wsem = (w0, w1)
        osem = (o0, o1)

        pltpu.sync_copy(pidx_hbm.at[pl.ds(tbase, tok_w)], pidx)
        pltpu.sync_copy(gap_hbm, gapt)

        def start_win(ci, b):
            vec = pidx[pl.ds(ci * CH, 16)]
            lo8 = pl.multiple_of(lax.shift_right_logical(vec[0], 16), 8)
            pltpu.async_copy(obs_hbm.at[pl.ds(lo8, WIN)], wins[b], wsem[b])

        def wait_win(b):
            pltpu.make_async_copy(obs_hbm.at[pl.ds(0, WIN)],
                                  wins[b], wsem[b]).wait()

        def wait_out(b):
            pltpu.make_async_copy(obuf[b], out_hbm.at[0, pl.ds(0, CH)],
                                  osem[b]).wait()

        start_win(0, 0)
        start_win(1, 1)

        def process(i, ci, b):
            wait_win(b)

            @pl.when(i > 0)
            def _():
                wait_out(b)

            c0 = pidx[pl.ds(ci * CH, 16)]
            c1 = pidx[pl.ds(ci * CH + 16, 16)]
            for j in range(CH):
                v = c0[j] if j < 16 else c1[j - 16]
                oloc = v & 63
                g = lax.shift_right_logical(v, 6) & 127
                wv = [wins[b][oloc, pl.ds(k * 16, 16)] for k in range(NLANE)]
                for k in range(NLANE):
                    obuf[b][j, pl.ds(k * 16, 16)] = wv[k]
                gv = [gapt[g, pl.ds(k * 16, 16)] for k in range(NLANE)]
                for k in range(NLANE):
                    obuf[b][j, pl.ds(DH + k * 16, 16)] = gv[k]
            dst = pl.multiple_of(t0 + ci * CH, 8)
            pltpu.async_copy(obuf[b], out_hbm.at[bi, pl.ds(dst, CH)], osem[b])

            @pl.when(ci + 2 < nch)
            def _():
                start_win(ci + 2, b)

        def body(i, _):
            process(i, 2 * i, 0)
            process(i, 2 * i + 1, 1)
            return 0

        lax.fori_loop(0, nch // 2, body, 0)
        wait_out(0)
        wait_out(1)

    return _expand_kernel


def kernel(x, padding_mask, obs_table, W1, b1, W2, b2):
    mask_i32 = (~padding_mask).astype(jnp.int32)
    pack, gap_tbl = _index_call(
        mask_i32, W1, b1.reshape(1, H), W2, b2.reshape(1, DH))
    return _make_expand_kernel()(obs_table, gap_tbl, pack.reshape(-1))


# speedup vs baseline: 5.5660x; 1.2938x over previous
import functools

import jax
import jax.numpy as jnp
from jax import lax
from jax.experimental import pallas as pl
from jax.experimental.pallas import tpu as pltpu
from jax.experimental.pallas import tpu_sc as plsc

B, T, D = 4, 8192, 768
H = D // 4
DH = D // 2
NLANE = DH // 16
MAX_OBS = 2000
GAP_ROWS = 104
N = B * T
CH = 32
WIN = 40

_NEG = -(2 ** 30)


def _erf(z):
    a1, a2, a3, a4, a5 = (0.254829592, -0.284496736, 1.421413741,
                          -1.453152027, 1.061405429)
    p = 0.3275911
    s = jnp.sign(z)
    az = jnp.abs(z)
    t = 1.0 / (1.0 + p * az)
    poly = ((((a5 * t + a4) * t + a3) * t + a2) * t + a1) * t
    return s * (1.0 - poly * jnp.exp(-az * az))


def _index_kernel(mask_ref, w1_ref, b1_ref, w2_ref, b2_ref,
                  pack_ref, tbl_ref):
    valid = mask_ref[...]
    csum = valid
    s = 1
    while s < T:
        shifted = jnp.concatenate(
            [jnp.zeros((B, s), jnp.int32), csum[:, :T - s]], axis=1)
        csum = csum + shifted
        s *= 2
    obs = jnp.clip(csum - 1, 0, MAX_OBS - 1)

    pos = lax.broadcasted_iota(jnp.int32, (B, T), 1)
    lastv = jnp.where(valid > 0, pos, _NEG)
    s = 1
    while s < T:
        shifted = jnp.concatenate(
            [jnp.full((B, s), _NEG, jnp.int32), lastv[:, :T - s]], axis=1)
        lastv = jnp.maximum(lastv, shifted)
        s *= 2
    gap = jnp.where(lastv < 0, 0, jnp.minimum(pos - lastv, 100))

    pos_in = pos & (CH - 1)
    f = jnp.where(pos_in == 0, obs, -1)
    s = 1
    while s < CH:
        shifted = jnp.concatenate(
            [jnp.full((B, s), -1, jnp.int32), f[:, :T - s]], axis=1)
        f = jnp.maximum(f, jnp.where(pos_in >= s, shifted, -1))
        s *= 2
    lo8 = jnp.minimum(f & -8, MAX_OBS - WIN)
    oloc = obs - lo8
    pack_ref[...] = oloc | (gap << 6) | (lo8 << 16)

    gmax = jnp.max(gap).astype(jnp.float32)
    k = lax.broadcasted_iota(jnp.int32, (GAP_ROWS, H), 0).astype(jnp.float32)
    g = k / (gmax + 1e-8)
    z = g * w1_ref[...] + b1_ref[...]
    h1 = 0.5 * z * (1.0 + _erf(z * 0.7071067811865476))
    tbl_ref[...] = (jnp.dot(h1, w2_ref[...], preferred_element_type=jnp.float32)
                    + b2_ref[...])


_index_call = pl.pallas_call(
    _index_kernel,
    out_shape=(
        jax.ShapeDtypeStruct((B, T), jnp.int32),
        jax.ShapeDtypeStruct((GAP_ROWS, DH), jnp.float32),
    ),
)


@functools.lru_cache(maxsize=None)
def _make_expand_kernel():
    info = plsc.get_sparse_core_info()
    nc, ns = info.num_cores, info.num_subcores
    nw = nc * ns
    tok_w = N // nw
    nch = tok_w // CH
    mesh = plsc.VectorSubcoreMesh(core_axis_name="c", subcore_axis_name="s")

    wpb = T // tok_w

    @functools.partial(
        pl.kernel,
        mesh=mesh,
        out_type=jax.ShapeDtypeStruct((B, T, D), jnp.float32),
        scratch_types=[
            pltpu.VMEM((tok_w,), jnp.int32),
            pltpu.VMEM((GAP_ROWS, DH), jnp.float32),
            pltpu.VMEM((WIN, DH), jnp.float32),
            pltpu.VMEM((WIN, DH), jnp.float32),
            pltpu.VMEM((CH, D), jnp.float32),
            pltpu.VMEM((CH, D), jnp.float32),
            pltpu.SemaphoreType.DMA,
            pltpu.SemaphoreType.DMA,
            pltpu.SemaphoreType.DMA,
            pltpu.SemaphoreType.DMA,
        ],
    )
    def _expand_kernel(obs_hbm, gap_hbm, pidx_hbm, out_hbm,
                       pidx, gapt, win0, win1, ob0, ob1, w0, w1, o0, o1):
        wid = lax.axis_index("s") * nc + lax.axis_index("c")
        tbase = wid * tok_w
        bi = wid // wpb
        t0 = (wid % wpb) * tok_w
        wins = (win0, win1)
        obuf = (ob0, ob1)
        wsem = (w0, w1)
        osem = (o0, o1)

        pltpu.sync_copy(pidx_hbm.at[pl.ds(tbase, tok_w)], pidx)
        pltpu.sync_copy(gap_hbm, gapt)

        def start_win(ci, b):
            vec = pidx[pl.ds(ci * CH, 16)]
            lo8 = pl.multiple_of(lax.shift_right_logical(vec[0], 16), 8)
            pltpu.async_copy(obs_hbm.at[pl.ds(lo8, WIN)], wins[b], wsem[b])

        def wait_win(b):
            pltpu.make_async_copy(obs_hbm.at[pl.ds(0, WIN)],
                                  wins[b], wsem[b]).wait()

        def wait_out(b):
            pltpu.make_async_copy(obuf[b], out_hbm.at[0, pl.ds(0, CH)],
                                  osem[b]).wait()

        start_win(0, 0)
        start_win(1, 1)

        def process(i, ci, b):
            wait_win(b)

            @pl.when(i > 0)
            def _():
                wait_out(b)

            c0 = pidx[pl.ds(ci * CH, 16)]
            c1 = pidx[pl.ds(ci * CH + 16, 16)]
            gv = None
            for j in range(CH):
                v = c0[j] if j < 16 else c1[j - 16]
                oloc = v & 63
                g = lax.shift_right_logical(v, 6) & 127
                wv = []
                for k in range(NLANE):
                    wv.append(wins[b][oloc, pl.ds(k * 16, 16)])
                    if gv is not None:
                        obuf[b][j - 1, pl.ds(DH + k * 16, 16)] = gv[k]
                gv = []
                for k in range(NLANE):
                    gv.append(gapt[g, pl.ds(k * 16, 16)])
                    obuf[b][j, pl.ds(k * 16, 16)] = wv[k]
            for k in range(NLANE):
                obuf[b][CH - 1, pl.ds(DH + k * 16, 16)] = gv[k]
            dst = pl.multiple_of(t0 + ci * CH, 8)
            pltpu.async_copy(obuf[b], out_hbm.at[bi, pl.ds(dst, CH)], osem[b])

            @pl.when(ci + 2 < nch)
            def _():
                start_win(ci + 2, b)

        def body(i, _):
            process(i, 2 * i, 0)
            process(i, 2 * i + 1, 1)
            return 0

        lax.fori_loop(0, nch // 2, body, 0)
        wait_out(0)
        wait_out(1)

    return _expand_kernel


def kernel(x, padding_mask, obs_table, W1, b1, W2, b2):
    mask_i32 = (~padding_mask).astype(jnp.int32)
    pack, gap_tbl = _index_call(
        mask_i32, W1, b1.reshape(1, H), W2, b2.reshape(1, DH))
    return _make_expand_kernel()(obs_table, gap_tbl, pack.reshape(-1))
